# baseline (speedup 1.0000x reference)
import jax
import jax.numpy as jnp
from jax import lax
from jax.experimental import pallas as pl
from jax.experimental.pallas import tpu as pltpu
from jax.experimental.pallas import tpu_sc as plsc

B, H, D = 4096, 200, 64
NV = 1000
ROWS = B * H
NC, NS = 2, 16
NW = NC * NS
RPW = ROWS // NW
C = 128
IROWS = ROWS // C
IRPW = IROWS // NW
NBUF = 2
T = IRPW // NBUF


def _body(vidx_hbm, nidx_hbm, verbs_hbm, nouns_hbm, out_hbm,
          verbs_v, vidx_v, nidx_v, vb0, vb1, nb0, nb1,
          sem_i, sem_g, sem_wv, sem_wn):
    vbufs = [vb0, vb1]
    nbufs = [nb0, nb1]
    wid = lax.axis_index("s") * NC + lax.axis_index("c")
    row0 = wid * RPW
    irow0 = wid * IRPW

    pltpu.sync_copy(verbs_hbm, verbs_v)
    pltpu.sync_copy(vidx_hbm.at[pl.ds(wid * RPW, RPW)], vidx_v)
    pltpu.sync_copy(nidx_hbm.at[pl.ds(irow0, NBUF)], nidx_v.at[0])

    lanes = lax.iota(jnp.int32, 16)

    def fill_verbs(g, vbuf):
        idxs = [vidx_v[pl.ds(g * C + k * 16, 16)] for k in range(C // 16)]
        rows = [lanes + 16 * k for k in range(C // 16)]

        @plsc.parallel_loop(0, D, step=1, unroll=8)
        def _(w):
            ws = jnp.full((16,), 0, jnp.int32) + w
            for k in range(C // 16):
                vals = plsc.load_gather(verbs_v, [idxs[k], ws])
                plsc.store_scatter(vbuf, [rows[k], ws], vals)

    def wait_write_v(j):
        pltpu.make_async_copy(
            vbufs[j], out_hbm.at[pl.ds(0, C), pl.ds(0, D)], sem_wv.at[j]).wait()

    def wait_write_n(j):
        pltpu.make_async_copy(
            nbufs[j], out_hbm.at[pl.ds(0, C), pl.ds(D, D)], sem_wn.at[j]).wait()

    def block(t, carry):
        p = lax.rem(t, 2)
        pn = 1 - p

        @pl.when(t > 0)
        def _():
            pltpu.make_async_copy(
                nidx_hbm.at[pl.ds(irow0, NBUF)], nidx_v.at[p], sem_i.at[p]).wait()
            for j in range(NBUF):
                wait_write_n(j)

        gathers = []
        for j in range(NBUF):
            gathers.append(pltpu.async_copy(
                nouns_hbm.at[nidx_v.at[p, j]], nbufs[j], sem_g.at[j]))

        @pl.when(t + 1 < T)
        def _():
            pltpu.async_copy(
                nidx_hbm.at[pl.ds(irow0 + (t + 1) * NBUF, NBUF)],
                nidx_v.at[pn], sem_i.at[pn])

        for j in range(NBUF):
            @pl.when(t > 0)
            def _(j=j):
                wait_write_v(j)
            fill_verbs(t * NBUF + j, vbufs[j])

        for j in range(NBUF):
            g = t * NBUF + j
            r0 = row0 + g * C
            gathers[j].wait()
            pltpu.async_copy(
                vbufs[j], out_hbm.at[pl.ds(r0, C), pl.ds(0, D)], sem_wv.at[j])
            pltpu.async_copy(
                nbufs[j], out_hbm.at[pl.ds(r0, C), pl.ds(D, D)], sem_wn.at[j])
        return carry

    lax.fori_loop(0, T, block, 0)
    for j in range(NBUF):
        wait_write_v(j)
        wait_write_n(j)


@jax.jit
def _run(vidx, nidx, verbs_table, nouns_table):
    fn = pl.kernel(
        _body,
        out_type=jax.ShapeDtypeStruct((ROWS, 2 * D), jnp.float32),
        mesh=plsc.VectorSubcoreMesh(core_axis_name="c", subcore_axis_name="s"),
        compiler_params=pltpu.CompilerParams(
            use_tc_tiling_on_sc=False, needs_layout_passes=False),
        scratch_types=(
            [
                pltpu.VMEM((NV, D), jnp.float32),
                pltpu.VMEM((RPW,), jnp.int32),
                pltpu.VMEM((2, NBUF, C), jnp.int32),
            ]
            + [pltpu.VMEM((C, D), jnp.float32)] * (2 * NBUF)
            + [pltpu.SemaphoreType.DMA((2,))]
            + [pltpu.SemaphoreType.DMA((NBUF,))] * 3
        ),
    )
    return fn(vidx, nidx, verbs_table, nouns_table)


def kernel(observed_labels, verbs_table, nouns_table):
    idx = observed_labels.reshape(ROWS, 2).T
    vidx = idx[0].reshape(ROWS)
    nidx = idx[1].reshape(IROWS, C)
    out = _run(vidx, nidx, verbs_table, nouns_table)
    return out.reshape(B, H, 2 * D)
